# baseline (speedup 1.0000x reference)
import functools

import jax
import jax.numpy as jnp
from jax import lax
from jax.experimental import pallas as pl
from jax.experimental.pallas import tpu as pltpu
from jax.experimental.pallas import tpu_sc as plsc

CHUNK = 1600
BAND = CHUNK // 4
SUBS = [(0, 128), (128, 128), (256, 128), (384, 16)]


def _make_gather(total: int, n_dim: int):
    info = plsc.get_sparse_core_info()
    nc, ns = info.num_cores, info.num_subcores
    nw = nc * ns
    per_w = total // nw
    n_chunks = per_w // CHUNK
    n_pairs = n_chunks // 2
    pack = 128 // n_dim
    out_rows = total // pack

    mesh = plsc.VectorSubcoreMesh(core_axis_name="c", subcore_axis_name="s")

    @functools.partial(
        pl.kernel,
        mesh=mesh,
        out_type=jax.ShapeDtypeStruct((out_rows, 128), jnp.float32),
        scratch_types=[
            pltpu.VMEM((2 * CHUNK,), jnp.int32),
            pltpu.VMEM((pack, BAND, n_dim), jnp.float32),
            pltpu.VMEM((pack, BAND, n_dim), jnp.float32),
            pltpu.SemaphoreType.DMA,
            pltpu.SemaphoreType.DMA,
            pltpu.SemaphoreType.DMA,
            pltpu.SemaphoreType.DMA,
        ],
        compiler_params=pltpu.CompilerParams(use_tc_tiling_on_sc=False),
    )
    def gather_kernel(idx_hbm, table_hbm, out_hbm, idx_v, rows0, rows1,
                      semg0, semg1, semw0, semw1):
        wid = lax.axis_index("s") * nc + lax.axis_index("c")
        w_i0 = wid * per_w
        w_r0 = wid * (per_w // pack)

        def load_idx(pair):
            pltpu.sync_copy(idx_hbm.at[pl.ds(w_i0 + pair * 2 * CHUNK,
                                             2 * CHUNK)], idx_v)

        def fire_gathers(rows_v, sem, base):
            copies = []
            for q in range(pack):
                for off, n in SUBS:
                    copies.append(pltpu.async_copy(
                        table_hbm.at[idx_v.at[pl.ds(base + q * BAND + off, n)]],
                        rows_v.at[q, pl.ds(off, n)], sem))
            return copies

        def fire_writes(rows_v, sem, chunk):
            r0 = w_r0 + chunk * BAND
            for q in range(pack):
                pltpu.async_copy(
                    rows_v.at[q],
                    out_hbm.at[pl.ds(r0, BAND), pl.ds(q * n_dim, n_dim)], sem)

        def wait_writes(rows_v, sem):
            for q in range(pack):
                pltpu.make_async_copy(
                    rows_v.at[q],
                    out_hbm.at[pl.ds(0, BAND), pl.ds(q * n_dim, n_dim)],
                    sem).wait()

        load_idx(0)
        g0 = fire_gathers(rows0, semg0, 0)
        g1 = fire_gathers(rows1, semg1, CHUNK)
        for c in g0:
            c.wait()
        fire_writes(rows0, semw0, 0)
        for c in g1:
            c.wait()
        fire_writes(rows1, semw1, 1)

        def pair_body(p, carry):
            wait_writes(rows0, semw0)
            load_idx(p)
            g0 = fire_gathers(rows0, semg0, 0)
            wait_writes(rows1, semw1)
            g1 = fire_gathers(rows1, semg1, CHUNK)
            for c in g0:
                c.wait()
            fire_writes(rows0, semw0, 2 * p)
            for c in g1:
                c.wait()
            fire_writes(rows1, semw1, 2 * p + 1)
            return carry

        lax.fori_loop(1, n_pairs, pair_body, 0)
        wait_writes(rows0, semw0)
        wait_writes(rows1, semw1)

    return gather_kernel


def kernel(node_ids, emb_table):
    b, h = node_ids.shape
    n_nodes, n_dim = emb_table.shape
    total = b * h
    pack = 128 // n_dim
    idx1d = (node_ids.reshape(total // CHUNK, CHUNK // pack, pack)
             .transpose(0, 2, 1).reshape(total).astype(jnp.int32))
    out2d = _make_gather(total, n_dim)(idx1d, emb_table)
    return out2d.reshape(b, h, n_dim)
